# baseline (speedup 1.0000x reference)
import jax
import jax.numpy as jnp
from jax.experimental import pallas as pl
from jax.experimental.pallas import tpu as pltpu

_OUT_ACTIONS = 18
_BLOCK_B = 4096


def _mlp_kernel(x_ref, w1_ref, b1_ref, w2_ref, b2_ref, o_ref, *, block_b):
    i = pl.program_id(0)
    x = x_ref[...].astype(jnp.bfloat16)
    w1 = w1_ref[...].astype(jnp.bfloat16)
    h = jnp.dot(x, w1, preferred_element_type=jnp.float32)
    h = jnp.maximum(h + b1_ref[...], 0.0).astype(jnp.bfloat16)
    w2 = w2_ref[...].astype(jnp.bfloat16)
    y = jnp.dot(h, w2, preferred_element_type=jnp.float32)
    y = (y + b2_ref[...])[:, :_OUT_ACTIONS]
    o_ref[pl.ds(i * block_b, block_b), :] = y


@jax.jit
def kernel(x, w1, b1, w2, b2):
    import functools
    B, K = x.shape
    Hp = w1.shape[1]
    Np = w2.shape[1]
    block_b = min(_BLOCK_B, B)
    nb = pl.cdiv(B, block_b)
    flops = 2 * B * (K * Hp + Hp * Np)
    w_bytes = (w1.size + b1.size + w2.size + b2.size) * 4
    cost = pl.CostEstimate(
        flops=flops, transcendentals=0,
        bytes_accessed=B * K * 4 + w_bytes + B * _OUT_ACTIONS * 4)
    return pl.pallas_call(
        functools.partial(_mlp_kernel, block_b=block_b),
        out_shape=jax.ShapeDtypeStruct((B, _OUT_ACTIONS), jnp.float32),
        grid=(nb,),
        in_specs=[
            pl.BlockSpec((block_b, K), lambda i: (i, 0)),
            pl.BlockSpec((K, Hp), lambda i: (0, 0)),
            pl.BlockSpec((1, Hp), lambda i: (0, 0)),
            pl.BlockSpec((Hp, Np), lambda i: (0, 0)),
            pl.BlockSpec((1, Np), lambda i: (0, 0)),
        ],
        out_specs=pl.BlockSpec(memory_space=pltpu.MemorySpace.VMEM),
        compiler_params=pltpu.CompilerParams(
            dimension_semantics=("arbitrary",)),
        cost_estimate=cost,
    )(x, w1, b1, w2, b2)


# speedup vs baseline: 1.0577x; 1.0577x over previous
import jax
import jax.numpy as jnp
from jax.experimental import pallas as pl
from jax.experimental.pallas import tpu as pltpu

_OUT_ACTIONS = 18
_BLOCK_B = 4096


def _mlp_kernel(x_ref, w1_ref, b1_ref, w2_ref, b2_ref, o_ref):
    x = x_ref[...].astype(jnp.bfloat16)
    w1 = w1_ref[...].astype(jnp.bfloat16)
    h = jnp.dot(x, w1, preferred_element_type=jnp.float32)
    h = jnp.maximum(h + b1_ref[...], 0.0).astype(jnp.bfloat16)
    w2 = w2_ref[...].astype(jnp.bfloat16)
    y = jnp.dot(h, w2, preferred_element_type=jnp.float32)
    y = y + b2_ref[...]
    o_ref[...] = y[:, :_OUT_ACTIONS]


@jax.jit
def kernel(x, w1, b1, w2, b2):
    B, K = x.shape
    Hp = w1.shape[1]
    Np = w2.shape[1]
    block_b = min(_BLOCK_B, B)
    nb = pl.cdiv(B, block_b)
    flops = 2 * B * (K * Hp + Hp * Np)
    w_bytes = (w1.size + b1.size + w2.size + b2.size) * 4
    cost = pl.CostEstimate(
        flops=flops, transcendentals=0,
        bytes_accessed=B * K * 4 + w_bytes + B * _OUT_ACTIONS * 4)
    return pl.pallas_call(
        _mlp_kernel,
        out_shape=jax.ShapeDtypeStruct((B, _OUT_ACTIONS), jnp.float32),
        grid=(nb,),
        in_specs=[
            pl.BlockSpec((block_b, K), lambda i: (i, 0)),
            pl.BlockSpec((K, Hp), lambda i: (0, 0)),
            pl.BlockSpec((1, Hp), lambda i: (0, 0)),
            pl.BlockSpec((Hp, Np), lambda i: (0, 0)),
            pl.BlockSpec((1, Np), lambda i: (0, 0)),
        ],
        out_specs=pl.BlockSpec((block_b, _OUT_ACTIONS), lambda i: (i, 0)),
        compiler_params=pltpu.CompilerParams(
            dimension_semantics=("parallel",)),
        cost_estimate=cost,
    )(x, w1, b1, w2, b2)
